# baseline (speedup 1.0000x reference)
import functools

import jax
import jax.numpy as jnp
from jax import lax
from jax.experimental import pallas as pl
from jax.experimental.pallas import tpu as pltpu
from jax.experimental.pallas import tpu_sc as plsc

N = 10000
E = 320000
NSC = 2
NTEC = 16
NW = NSC * NTEC
EPW = E // NW
C = 80
CPT = EPW // C
NBUF = 4
EB = 2 * NBUF
CD = 2000
RPT = 640
PAD_N = NTEC * RPT
LASTR = N - (NTEC - 1) * RPT
ZR = C
RBLK = 1000

_mesh = functools.partial(
    plsc.VectorSubcoreMesh,
    core_axis_name="c", subcore_axis_name="s",
    num_cores=NSC, num_subcores=NTEC,
)


def _deg_body(src_hbm, dst_hbm, outs_hbm, outd_hbm, hs, hd, sidx, didx, sem):
    cid = lax.axis_index("c")
    sid = lax.axis_index("s")
    wid = cid * NTEC + sid
    zeros16 = jnp.zeros((16,), jnp.float32)
    ones16 = jnp.ones((16,), jnp.float32)

    def zero_body(r, carry):
        hs[pl.ds(r * 16, 16)] = zeros16
        hd[pl.ds(r * 16, 16)] = zeros16
        return carry

    lax.fori_loop(0, N // 16, zero_body, 0)

    def chunk_body(i, carry):
        base = wid * EPW + i * CD
        pltpu.sync_copy(src_hbm.at[pl.ds(base, CD)], sidx)
        pltpu.sync_copy(dst_hbm.at[pl.ds(base, CD)], didx)

        def lane_body(k, c2):
            s = sidx[pl.ds(k * 16, 16)]
            plsc.addupdate_scatter(hs, [s], ones16)
            d = didx[pl.ds(k * 16, 16)]
            plsc.addupdate_scatter(hd, [d], ones16)
            return c2

        return lax.fori_loop(0, CD // 16, lane_body, carry)

    lax.fori_loop(0, EPW // CD, chunk_body, 0)
    pltpu.sync_copy(hs, outs_hbm.at[wid])
    pltpu.sync_copy(hd, outd_hbm.at[wid])


@jax.jit
def _degree_hist(src, dst):
    k = pl.kernel(
        _deg_body,
        out_type=(
            jax.ShapeDtypeStruct((NW, N), jnp.float32),
            jax.ShapeDtypeStruct((NW, N), jnp.float32),
        ),
        mesh=_mesh(),
        scratch_types=[
            pltpu.VMEM((N,), jnp.float32),
            pltpu.VMEM((N,), jnp.float32),
            pltpu.VMEM((CD,), jnp.int32),
            pltpu.VMEM((CD,), jnp.int32),
            pltpu.SemaphoreType.DMA,
        ],
        compiler_params=pltpu.CompilerParams(needs_layout_passes=False),
    )
    return k(src, dst)


def _spmm_body(h_hbm, eidx_hbm, out_hbm, *refs):
    ebufs = refs[0:EB]
    rows = refs[EB:EB + NBUF]
    acc = refs[EB + NBUF]
    sems = refs[EB + NBUF + 1:]
    semE = sems[0:EB]
    semG = sems[EB:EB + NBUF]
    semS = sems[EB + NBUF:EB + 2 * NBUF]
    F = rows[0].shape[1]
    cid = lax.axis_index("c")
    sid = lax.axis_index("s")
    wid = cid * NTEC + sid
    zeros32 = jnp.zeros((32,), jnp.bfloat16)

    def zfill(r, carry):
        for k in range(F // 32):
            rows[0][r, pl.ds(k * 32, 32)] = zeros32
        return carry

    lax.fori_loop(0, ZR, zfill, 0)

    row0 = pl.multiple_of(sid * RPT, 8)
    zdescs = [pltpu.async_copy(rows[0], acc.at[pl.ds(row0 + j * ZR, ZR)],
                               semS[0])
              for j in range(RPT // ZR)]
    for d in zdescs:
        d.wait()
    plsc.subcore_barrier()

    cbase = wid * CPT
    edescs = [pltpu.async_copy(eidx_hbm.at[cbase + b], ebufs[b], semE[b])
              for b in range(NBUF)]
    for b in range(NBUF - 1):
        edescs[b].wait()
        pltpu.async_copy(h_hbm.at[ebufs[b].at[0]], rows[b], semG[b])

    def group_body(g, carry):
        for u in range(EB):
            j = g * EB + u
            b = u % NBUF
            e = u % EB

            @pl.when(j < CPT)
            def _():
                pltpu.make_async_copy(h_hbm.at[ebufs[e].at[0]], rows[b],
                                      semG[b]).wait()
                pltpu.async_copy(rows[b], acc.at[ebufs[e].at[1]], semS[b],
                                 add=True)

            @pl.when(j + NBUF < CPT)
            def _():
                e2 = (u + NBUF) % EB
                pltpu.async_copy(eidx_hbm.at[cbase + j + NBUF],
                                 ebufs[e2], semE[e2])

            k = j + NBUF - 1
            bk = (u + NBUF - 1) % NBUF
            ek = (u + NBUF - 1) % EB

            @pl.when((k < CPT) & (j >= 1))
            def _():
                pltpu.make_async_copy(rows[bk], acc.at[ebufs[ek].at[1]],
                                      semS[bk]).wait()

            @pl.when(k < CPT)
            def _():
                pltpu.make_async_copy(eidx_hbm.at[0], ebufs[ek],
                                      semE[ek]).wait()
                pltpu.async_copy(h_hbm.at[ebufs[ek].at[0]], rows[bk],
                                 semG[bk])

        return carry

    lax.fori_loop(0, (CPT + EB - 1) // EB, group_body, 0)
    for r in range(NBUF):
        j = CPT - NBUF + r
        b = j % NBUF
        e = j % EB
        pltpu.make_async_copy(rows[b], acc.at[ebufs[e].at[1]],
                              semS[b]).wait()
    plsc.subcore_barrier()
    obase = pl.multiple_of(cid * N + row0, 8)

    @pl.when(sid < NTEC - 1)
    def _():
        pltpu.sync_copy(acc.at[pl.ds(row0, RPT)], out_hbm.at[pl.ds(obase, RPT)])

    @pl.when(sid == NTEC - 1)
    def _():
        pltpu.sync_copy(acc.at[pl.ds(row0, LASTR)],
                        out_hbm.at[pl.ds(obase, LASTR)])


@functools.partial(jax.jit, static_argnames=("F",))
def _spmm(h, eidx, F):
    k = pl.kernel(
        _spmm_body,
        out_type=jax.ShapeDtypeStruct((NSC * N, F), jnp.bfloat16),
        mesh=_mesh(),
        scratch_types=(
            [pltpu.VMEM((2, C), jnp.int32) for _ in range(EB)]
            + [pltpu.VMEM((C, F), jnp.bfloat16) for _ in range(NBUF)]
            + [pltpu.VMEM_SHARED((PAD_N, F), jnp.bfloat16)]
            + [pltpu.SemaphoreType.DMA for _ in range(EB + 2 * NBUF)]
        ),
        compiler_params=pltpu.CompilerParams(
            needs_layout_passes=False,
            use_tc_tiling_on_sc=False,
        ),
    )
    return k(h, eidx)


_DOT = dict(preferred_element_type=jnp.float32,
            precision=lax.Precision.HIGHEST)


def _prep_body(hs_ref, hd_ref, x_ref, xs_ref, rsq_ref):
    ones = jnp.ones((NW, 1), jnp.float32)
    dims = (((0,), (0,)), ((), ()))
    so = lax.dot_general(hs_ref[...], ones, dims, **_DOT)
    si = lax.dot_general(hd_ref[...], ones, dims, **_DOT)
    ro = lax.rsqrt(jnp.maximum(so, 1.0))
    ri = lax.rsqrt(jnp.maximum(si, 1.0))
    rsq_ref[...] = jnp.concatenate([ro, ri], axis=1)
    xs_ref[...] = (x_ref[...] * ro).astype(jnp.bfloat16)


@jax.jit
def _prep(hs, hd, x):
    return pl.pallas_call(
        _prep_body,
        out_shape=[
            jax.ShapeDtypeStruct((N, 128), jnp.bfloat16),
            jax.ShapeDtypeStruct((N, 2), jnp.float32),
        ],
    )(hs, hd, x)


def _fuse1_body(p_ref, rsq_ref, w1_ref, b1_ref, w2_ref, o_ref):
    p = p_ref[0].astype(jnp.float32) + p_ref[1].astype(jnp.float32)
    agg = p * rsq_ref[:, 1:2]
    h = jax.nn.relu(jnp.dot(agg, w1_ref[...], **_DOT) + b1_ref[...])
    o_ref[...] = jnp.dot(h * rsq_ref[:, 0:1], w2_ref[...],
                         **_DOT).astype(jnp.bfloat16)


@jax.jit
def _fuse1(p, rsq, W1, b1, W2):
    return pl.pallas_call(
        _fuse1_body,
        grid=(N // RBLK,),
        in_specs=[
            pl.BlockSpec((2, RBLK, 128), lambda i: (0, i, 0)),
            pl.BlockSpec((RBLK, 2), lambda i: (i, 0)),
            pl.BlockSpec((128, 256), lambda i: (0, 0)),
            pl.BlockSpec((1, 256), lambda i: (0, 0)),
            pl.BlockSpec((256, 128), lambda i: (0, 0)),
        ],
        out_specs=pl.BlockSpec((RBLK, 128), lambda i: (i, 0)),
        out_shape=jax.ShapeDtypeStruct((N, 128), jnp.bfloat16),
    )(p, rsq, W1, b1, W2)


def _fuse2_body(p_ref, rsq_ref, b2_ref, w3_ref, o_ref):
    p = p_ref[0].astype(jnp.float32) + p_ref[1].astype(jnp.float32)
    h = jax.nn.relu(p * rsq_ref[:, 1:2] + b2_ref[...])
    o_ref[...] = jnp.dot(h * rsq_ref[:, 0:1], w3_ref[...],
                         **_DOT).astype(jnp.bfloat16)


@jax.jit
def _fuse2(p, rsq, b2, W3):
    return pl.pallas_call(
        _fuse2_body,
        grid=(N // RBLK,),
        in_specs=[
            pl.BlockSpec((2, RBLK, 128), lambda i: (0, i, 0)),
            pl.BlockSpec((RBLK, 2), lambda i: (i, 0)),
            pl.BlockSpec((1, 128), lambda i: (0, 0)),
            pl.BlockSpec((128, 64), lambda i: (0, 0)),
        ],
        out_specs=pl.BlockSpec((RBLK, 64), lambda i: (i, 0)),
        out_shape=jax.ShapeDtypeStruct((N, 64), jnp.bfloat16),
    )(p, rsq, b2, W3)


def _final_body(p_ref, rsq_ref, b3_ref, wfc_ref, bfc_ref, o_ref, acc_ref):
    i = pl.program_id(0)
    p = p_ref[0].astype(jnp.float32) + p_ref[1].astype(jnp.float32)
    h = jax.nn.relu(p * rsq_ref[:, 1:2] + b3_ref[...])
    ps = jnp.sum(h, axis=0, keepdims=True)

    @pl.when(i == 0)
    def _():
        acc_ref[...] = jnp.zeros_like(acc_ref)

    acc_ref[...] += ps

    @pl.when(i == pl.num_programs(0) - 1)
    def _():
        hg = acc_ref[...] * (1.0 / N)
        o_ref[...] = jnp.dot(hg, wfc_ref[...], **_DOT) + bfc_ref[...]


@jax.jit
def _final(p, rsq, b3, Wfc, bfc):
    return pl.pallas_call(
        _final_body,
        grid=(N // RBLK,),
        in_specs=[
            pl.BlockSpec((2, RBLK, 64), lambda i: (0, i, 0)),
            pl.BlockSpec((RBLK, 2), lambda i: (i, 0)),
            pl.BlockSpec((1, 64), lambda i: (0, 0)),
            pl.BlockSpec((64, 2), lambda i: (0, 0)),
            pl.BlockSpec((1, 2), lambda i: (0, 0)),
        ],
        out_specs=pl.BlockSpec((1, 2), lambda i: (0, 0)),
        out_shape=jax.ShapeDtypeStruct((1, 2), jnp.float32),
        scratch_shapes=[pltpu.VMEM((1, 64), jnp.float32)],
    )(p, rsq, b3, Wfc, bfc)


def kernel(x, edge_index, W1, b1, W2, b2, W3, b3, Wfc, bfc):
    src = edge_index[0]
    dst = edge_index[1]
    eidx = jnp.transpose(edge_index.reshape(2, E // C, C), (1, 0, 2))
    hs, hd = _degree_hist(src, dst)
    xs, rsq = _prep(hs, hd, x)
    p1 = _spmm(xs, eidx, 128).reshape(NSC, N, 128)
    z2 = _fuse1(p1, rsq, W1, b1.reshape(1, -1), W2)
    p2 = _spmm(z2, eidx, 128).reshape(NSC, N, 128)
    z3 = _fuse2(p2, rsq, b2.reshape(1, -1), W3)
    p3 = _spmm(z3, eidx, 64).reshape(NSC, N, 64)
    return _final(p3, rsq, b3.reshape(1, -1), Wfc, bfc.reshape(1, -1))


# speedup vs baseline: 1.1093x; 1.1093x over previous
import functools

import jax
import jax.numpy as jnp
from jax import lax
from jax.experimental import pallas as pl
from jax.experimental.pallas import tpu as pltpu
from jax.experimental.pallas import tpu_sc as plsc

N = 10000
E = 320000
NSC = 2
NTEC = 16
NW = NSC * NTEC
EPW = E // NW
C = 80
CPT = EPW // C
NBUF = 4
EB = 2 * NBUF
CD = 2000
RPT = 640
PAD_N = NTEC * RPT
LASTR = N - (NTEC - 1) * RPT
ZR = C
RBLK = 1000

_mesh = functools.partial(
    plsc.VectorSubcoreMesh,
    core_axis_name="c", subcore_axis_name="s",
    num_cores=NSC, num_subcores=NTEC,
)


NPT = PAD_N // NTEC


def _deg_body(src_hbm, dst_hbm, deg_hbm, hs, hd, sidx, didx, shist, tbuf,
              res, sem):
    cid = lax.axis_index("c")
    sid = lax.axis_index("s")
    wid = cid * NTEC + sid
    zeros16 = jnp.zeros((16,), jnp.float32)
    ones16 = jnp.ones((16,), jnp.float32)

    def zero_body(r, carry):
        hs[pl.ds(r * 16, 16)] = zeros16
        hd[pl.ds(r * 16, 16)] = zeros16
        return carry

    lax.fori_loop(0, PAD_N // 16, zero_body, 0)

    def chunk_body(i, carry):
        base = wid * EPW + i * CD
        pltpu.sync_copy(src_hbm.at[pl.ds(base, CD)], sidx)
        pltpu.sync_copy(dst_hbm.at[pl.ds(base, CD)], didx)

        def lane_body(k, c2):
            s = sidx[pl.ds(k * 16, 16)]
            plsc.addupdate_scatter(hs, [s], ones16)
            d = didx[pl.ds(k * 16, 16)]
            plsc.addupdate_scatter(hd, [d], ones16)
            return c2

        return lax.fori_loop(0, CD // 16, lane_body, carry)

    lax.fori_loop(0, EPW // CD, chunk_body, 0)
    pltpu.sync_copy(hs, shist.at[sid, 0])
    pltpu.sync_copy(hd, shist.at[sid, 1])
    plsc.subcore_barrier()
    rbase = pl.multiple_of(sid * NPT, 8)
    pltpu.sync_copy(shist.at[:, :, pl.ds(rbase, NPT)], tbuf)

    def red_body(v, carry):
        for c in range(2):
            a = tbuf[0, c, pl.ds(v * 16, 16)]
            for t in range(1, NTEC):
                a = a + tbuf[t, c, pl.ds(v * 16, 16)]
            res[c, pl.ds(v * 16, 16)] = a
        return carry

    lax.fori_loop(0, NPT // 16, red_body, 0)
    pltpu.sync_copy(res, deg_hbm.at[pl.ds(4 * cid, 2), pl.ds(rbase, NPT)])

    def zres_body(v, carry):
        for c in range(2):
            res[c, pl.ds(v * 16, 16)] = zeros16
        return carry

    lax.fori_loop(0, NPT // 16, zres_body, 0)
    pltpu.sync_copy(res, deg_hbm.at[pl.ds(4 * cid + 2, 2), pl.ds(rbase, NPT)])


@jax.jit
def _degree_hist(src, dst):
    k = pl.kernel(
        _deg_body,
        out_type=jax.ShapeDtypeStruct((8, PAD_N), jnp.float32),
        mesh=_mesh(),
        scratch_types=[
            pltpu.VMEM((PAD_N,), jnp.float32),
            pltpu.VMEM((PAD_N,), jnp.float32),
            pltpu.VMEM((CD,), jnp.int32),
            pltpu.VMEM((CD,), jnp.int32),
            pltpu.VMEM_SHARED((NTEC, 2, PAD_N), jnp.float32),
            pltpu.VMEM((NTEC, 2, NPT), jnp.float32),
            pltpu.VMEM((2, NPT), jnp.float32),
            pltpu.SemaphoreType.DMA,
        ],
        compiler_params=pltpu.CompilerParams(
            needs_layout_passes=False,
            use_tc_tiling_on_sc=False,
        ),
    )
    return k(src, dst)


def _spmm_body(h_hbm, eidx_hbm, out_hbm, *refs):
    ebufs = refs[0:EB]
    rows = refs[EB:EB + NBUF]
    acc = refs[EB + NBUF]
    sems = refs[EB + NBUF + 1:]
    semE = sems[0:EB]
    semG = sems[EB:EB + NBUF]
    semS = sems[EB + NBUF:EB + 2 * NBUF]
    F = rows[0].shape[1]
    cid = lax.axis_index("c")
    sid = lax.axis_index("s")
    wid = cid * NTEC + sid
    zeros32 = jnp.zeros((32,), jnp.bfloat16)

    def zfill(r, carry):
        for k in range(F // 32):
            rows[0][r, pl.ds(k * 32, 32)] = zeros32
        return carry

    lax.fori_loop(0, ZR, zfill, 0)

    row0 = pl.multiple_of(sid * RPT, 8)
    zdescs = [pltpu.async_copy(rows[0], acc.at[pl.ds(row0 + j * ZR, ZR)],
                               semS[0])
              for j in range(RPT // ZR)]
    for d in zdescs:
        d.wait()
    plsc.subcore_barrier()

    cbase = wid * CPT
    edescs = [pltpu.async_copy(eidx_hbm.at[cbase + b], ebufs[b], semE[b])
              for b in range(NBUF)]
    for b in range(NBUF - 1):
        edescs[b].wait()
        pltpu.async_copy(h_hbm.at[ebufs[b].at[0]], rows[b], semG[b])

    def group_body(g, carry):
        for u in range(EB):
            j = g * EB + u
            b = u % NBUF
            e = u % EB

            @pl.when(j < CPT)
            def _():
                pltpu.make_async_copy(h_hbm.at[ebufs[e].at[0]], rows[b],
                                      semG[b]).wait()
                pltpu.async_copy(rows[b], acc.at[ebufs[e].at[1]], semS[b],
                                 add=True)

            @pl.when(j + NBUF < CPT)
            def _():
                e2 = (u + NBUF) % EB
                pltpu.async_copy(eidx_hbm.at[cbase + j + NBUF],
                                 ebufs[e2], semE[e2])

            k = j + NBUF - 1
            bk = (u + NBUF - 1) % NBUF
            ek = (u + NBUF - 1) % EB

            @pl.when((k < CPT) & (j >= 1))
            def _():
                pltpu.make_async_copy(rows[bk], acc.at[ebufs[ek].at[1]],
                                      semS[bk]).wait()

            @pl.when(k < CPT)
            def _():
                pltpu.make_async_copy(eidx_hbm.at[0], ebufs[ek],
                                      semE[ek]).wait()
                pltpu.async_copy(h_hbm.at[ebufs[ek].at[0]], rows[bk],
                                 semG[bk])

        return carry

    lax.fori_loop(0, (CPT + EB - 1) // EB, group_body, 0)
    for r in range(NBUF):
        j = CPT - NBUF + r
        b = j % NBUF
        e = j % EB
        pltpu.make_async_copy(rows[b], acc.at[ebufs[e].at[1]],
                              semS[b]).wait()
    plsc.subcore_barrier()

    @pl.when(sid < NTEC - 1)
    def _():
        pltpu.sync_copy(acc.at[pl.ds(row0, RPT)],
                        out_hbm.at[cid, pl.ds(row0, RPT)])

    @pl.when(sid == NTEC - 1)
    def _():
        pltpu.sync_copy(acc.at[pl.ds(row0, LASTR)],
                        out_hbm.at[cid, pl.ds(row0, LASTR)])


@functools.partial(jax.jit, static_argnames=("F",))
def _spmm(h, eidx, F):
    k = pl.kernel(
        _spmm_body,
        out_type=jax.ShapeDtypeStruct((NSC, N, F), jnp.bfloat16),
        mesh=_mesh(),
        scratch_types=(
            [pltpu.VMEM((2, C), jnp.int32) for _ in range(EB)]
            + [pltpu.VMEM((C, F), jnp.bfloat16) for _ in range(NBUF)]
            + [pltpu.VMEM_SHARED((PAD_N, F), jnp.bfloat16)]
            + [pltpu.SemaphoreType.DMA for _ in range(EB + 2 * NBUF)]
        ),
        compiler_params=pltpu.CompilerParams(
            needs_layout_passes=False,
            use_tc_tiling_on_sc=False,
        ),
    )
    return k(h, eidx)


_DOT = dict(preferred_element_type=jnp.float32,
            precision=lax.Precision.HIGHEST)


def _prep_body(deg_ref, x_ref, xs_ref, rsq_ref):
    ri = lax.broadcasted_iota(jnp.int32, (8, 2), 0)
    ci = lax.broadcasted_iota(jnp.int32, (8, 2), 1)
    sel = ((ri % 4) == ci).astype(jnp.float32)
    dims = (((0,), (0,)), ((), ()))
    s = lax.dot_general(deg_ref[...], sel, dims, **_DOT)
    r = lax.rsqrt(jnp.maximum(s, 1.0))[0:N, :]
    rsq_ref[...] = r
    xs_ref[...] = (x_ref[...] * r[:, 0:1]).astype(jnp.bfloat16)


@jax.jit
def _prep(deg, x):
    return pl.pallas_call(
        _prep_body,
        out_shape=[
            jax.ShapeDtypeStruct((N, 128), jnp.bfloat16),
            jax.ShapeDtypeStruct((N, 2), jnp.float32),
        ],
    )(deg, x)


def _fuse1_body(p_ref, rsq_ref, w1_ref, b1_ref, w2_ref, o_ref):
    p = p_ref[0].astype(jnp.float32) + p_ref[1].astype(jnp.float32)
    agg = (p * rsq_ref[:, 1:2]).astype(jnp.bfloat16)
    h = jax.nn.relu(jnp.dot(agg, w1_ref[...],
                            preferred_element_type=jnp.float32)
                    + b1_ref[...])
    hb = (h * rsq_ref[:, 0:1]).astype(jnp.bfloat16)
    o_ref[...] = jnp.dot(hb, w2_ref[...],
                         preferred_element_type=jnp.float32
                         ).astype(jnp.bfloat16)


@jax.jit
def _fuse1(p, rsq, W1, b1, W2):
    return pl.pallas_call(
        _fuse1_body,
        grid=(N // RBLK,),
        in_specs=[
            pl.BlockSpec((2, RBLK, 128), lambda i: (0, i, 0)),
            pl.BlockSpec((RBLK, 2), lambda i: (i, 0)),
            pl.BlockSpec((128, 256), lambda i: (0, 0)),
            pl.BlockSpec((1, 256), lambda i: (0, 0)),
            pl.BlockSpec((256, 128), lambda i: (0, 0)),
        ],
        out_specs=pl.BlockSpec((RBLK, 128), lambda i: (i, 0)),
        out_shape=jax.ShapeDtypeStruct((N, 128), jnp.bfloat16),
    )(p, rsq, W1, b1, W2)


def _fuse2_body(p_ref, rsq_ref, b2_ref, w3_ref, o_ref):
    p = p_ref[0].astype(jnp.float32) + p_ref[1].astype(jnp.float32)
    h = jax.nn.relu(p * rsq_ref[:, 1:2] + b2_ref[...])
    hb = (h * rsq_ref[:, 0:1]).astype(jnp.bfloat16)
    o_ref[...] = jnp.dot(hb, w3_ref[...],
                         preferred_element_type=jnp.float32
                         ).astype(jnp.bfloat16)


@jax.jit
def _fuse2(p, rsq, b2, W3):
    return pl.pallas_call(
        _fuse2_body,
        grid=(N // RBLK,),
        in_specs=[
            pl.BlockSpec((2, RBLK, 128), lambda i: (0, i, 0)),
            pl.BlockSpec((RBLK, 2), lambda i: (i, 0)),
            pl.BlockSpec((1, 128), lambda i: (0, 0)),
            pl.BlockSpec((128, 64), lambda i: (0, 0)),
        ],
        out_specs=pl.BlockSpec((RBLK, 64), lambda i: (i, 0)),
        out_shape=jax.ShapeDtypeStruct((N, 64), jnp.bfloat16),
    )(p, rsq, b2, W3)


def _final_body(p_ref, rsq_ref, b3_ref, wfc_ref, bfc_ref, o_ref, acc_ref):
    i = pl.program_id(0)
    p = p_ref[0].astype(jnp.float32) + p_ref[1].astype(jnp.float32)
    h = jax.nn.relu(p * rsq_ref[:, 1:2] + b3_ref[...])
    ps = jnp.sum(h, axis=0, keepdims=True)

    @pl.when(i == 0)
    def _():
        acc_ref[...] = jnp.zeros_like(acc_ref)

    acc_ref[...] += ps

    @pl.when(i == pl.num_programs(0) - 1)
    def _():
        hg = acc_ref[...] * (1.0 / N)
        o_ref[...] = jnp.dot(hg, wfc_ref[...], **_DOT) + bfc_ref[...]


@jax.jit
def _final(p, rsq, b3, Wfc, bfc):
    return pl.pallas_call(
        _final_body,
        grid=(N // RBLK,),
        in_specs=[
            pl.BlockSpec((2, RBLK, 64), lambda i: (0, i, 0)),
            pl.BlockSpec((RBLK, 2), lambda i: (i, 0)),
            pl.BlockSpec((1, 64), lambda i: (0, 0)),
            pl.BlockSpec((64, 2), lambda i: (0, 0)),
            pl.BlockSpec((1, 2), lambda i: (0, 0)),
        ],
        out_specs=pl.BlockSpec((1, 2), lambda i: (0, 0)),
        out_shape=jax.ShapeDtypeStruct((1, 2), jnp.float32),
        scratch_shapes=[pltpu.VMEM((1, 64), jnp.float32)],
    )(p, rsq, b3, Wfc, bfc)


def kernel(x, edge_index, W1, b1, W2, b2, W3, b3, Wfc, bfc):
    src = edge_index[0]
    dst = edge_index[1]
    eidx = jnp.concatenate(
        [edge_index[0].reshape(E // C, 1, C),
         edge_index[1].reshape(E // C, 1, C)], axis=1)
    deg = _degree_hist(src, dst)
    xs, rsq = _prep(deg, x)
    p1 = _spmm(xs, eidx, 128)
    z2 = _fuse1(p1, rsq, W1.astype(jnp.bfloat16), b1.reshape(1, -1),
                W2.astype(jnp.bfloat16))
    p2 = _spmm(z2, eidx, 128)
    z3 = _fuse2(p2, rsq, b2.reshape(1, -1), W3.astype(jnp.bfloat16))
    p3 = _spmm(z3, eidx, 64)
    return _final(p3, rsq, b3.reshape(1, -1), Wfc, bfc.reshape(1, -1))
